# baseline (speedup 1.0000x reference)
import jax
import jax.numpy as jnp
from jax import lax
from jax.experimental import pallas as pl
from jax.experimental.pallas import tpu as pltpu

_H = 4096
_E = 64
_BM = 512
_NBUF = 3


def _router_kernel(x_hbm, w_ref, o_hbm):
    def body(x_ref, o_ref):
        o_ref[...] = lax.dot_general(
            x_ref[...],
            w_ref[...],
            (((1,), (1,)), ((), ())),
            preferred_element_type=jnp.float32,
        )

    m = x_hbm.shape[0]
    pipeline = pltpu.emit_pipeline(
        body,
        grid=(m // _BM,),
        in_specs=[
            pl.BlockSpec(
                (_BM, _H),
                lambda i: (i, 0),
                pipeline_mode=pl.Buffered(buffer_count=_NBUF, use_lookahead=True),
            ),
        ],
        out_specs=[
            pl.BlockSpec((_BM, _E), lambda i: (i, 0)),
        ],
    )
    pipeline(x_hbm, o_hbm)


@jax.jit
def kernel(hidden_states, weight, bias):
    x = hidden_states.reshape(-1, _H)
    m = x.shape[0]
    out = pl.pallas_call(
        _router_kernel,
        in_specs=[
            pl.BlockSpec(memory_space=pl.ANY),
            pl.BlockSpec(memory_space=pltpu.VMEM),
        ],
        out_specs=pl.BlockSpec(memory_space=pl.ANY),
        out_shape=jax.ShapeDtypeStruct((m, _E), jnp.float32),
    )(x, weight)
    return out + bias[None, :]


# speedup vs baseline: 1.1210x; 1.1210x over previous
import jax
import jax.numpy as jnp
from jax import lax
from jax.experimental import pallas as pl
from jax.experimental.pallas import tpu as pltpu

_H = 4096
_E = 64
_BM = 1024
_NBUF = 3


def _router_kernel(x_hbm, w_ref, b_ref, o_hbm):
    def body(x_ref, o_ref):
        acc = lax.dot_general(
            x_ref[...],
            w_ref[...],
            (((1,), (1,)), ((), ())),
            preferred_element_type=jnp.float32,
        )
        o_ref[...] = acc + b_ref[...]

    m = x_hbm.shape[0]
    pipeline = pltpu.emit_pipeline(
        body,
        grid=(m // _BM,),
        in_specs=[
            pl.BlockSpec(
                (_BM, _H),
                lambda i: (i, 0),
                pipeline_mode=pl.Buffered(buffer_count=_NBUF, use_lookahead=True),
            ),
        ],
        out_specs=[
            pl.BlockSpec((_BM, _E), lambda i: (i, 0)),
        ],
    )
    pipeline(x_hbm, o_hbm)


@jax.jit
def kernel(hidden_states, weight, bias):
    x = hidden_states.reshape(-1, _H)
    m = x.shape[0]
    b2 = bias.reshape(1, _E)
    out = pl.pallas_call(
        _router_kernel,
        in_specs=[
            pl.BlockSpec(memory_space=pl.ANY),
            pl.BlockSpec(memory_space=pltpu.VMEM),
            pl.BlockSpec(memory_space=pltpu.VMEM),
        ],
        out_specs=pl.BlockSpec(memory_space=pl.ANY),
        out_shape=jax.ShapeDtypeStruct((m, _E), jnp.float32),
    )(x, weight, b2)
    return out


# speedup vs baseline: 1.1369x; 1.0142x over previous
import jax
import jax.numpy as jnp
from jax import lax
from jax.experimental import pallas as pl
from jax.experimental.pallas import tpu as pltpu

_H = 4096
_E = 64
_BM = 512
_NBUF = 3


def _router_kernel(x_hbm, w_ref, b_ref, o_hbm):
    def body(x_ref, o_ref):
        acc = lax.dot_general(
            x_ref[...],
            w_ref[...],
            (((1,), (1,)), ((), ())),
            preferred_element_type=jnp.float32,
        )
        o_ref[...] = acc + b_ref[...]

    m = x_hbm.shape[0]
    pipeline = pltpu.emit_pipeline(
        body,
        grid=(m // _BM,),
        in_specs=[
            pl.BlockSpec(
                (_BM, _H),
                lambda i: (i, 0),
                pipeline_mode=pl.Buffered(buffer_count=_NBUF, use_lookahead=True),
            ),
        ],
        out_specs=[
            pl.BlockSpec((_BM, _E), lambda i: (i, 0)),
        ],
    )
    pipeline(x_hbm, o_hbm)


@jax.jit
def kernel(hidden_states, weight, bias):
    x = hidden_states.reshape(-1, _H)
    m = x.shape[0]
    b2 = bias.reshape(1, _E)
    out = pl.pallas_call(
        _router_kernel,
        in_specs=[
            pl.BlockSpec(memory_space=pl.ANY),
            pl.BlockSpec(memory_space=pltpu.VMEM),
            pl.BlockSpec(memory_space=pltpu.VMEM),
        ],
        out_specs=pl.BlockSpec(memory_space=pl.ANY),
        out_shape=jax.ShapeDtypeStruct((m, _E), jnp.float32),
    )(x, weight, b2)
    return out
